# baseline (speedup 1.0000x reference)
import functools

import jax
import jax.numpy as jnp
from jax import lax
from jax.experimental import pallas as pl
from jax.experimental.pallas import tpu as pltpu
from jax.experimental.pallas import tpu_sc as plsc

NC = 2
NS = 16
NW = NC * NS
CH = 80


def _pad_nodes(n_nodes):
    block = NS * 128
    return ((n_nodes + block - 1) // block) * block


def _deg_partials(dst3, dst3t, ones, zeros, n_nodes):
    _, nfd, CHD = dst3.shape
    _, _, CHT = dst3t.shape
    NP = _pad_nodes(n_nodes)
    RPT = NP // NS
    W = 128
    mesh = plsc.VectorSubcoreMesh(core_axis_name="c", subcore_axis_name="s",
                                  num_cores=NC, num_subcores=NS)

    @functools.partial(
        pl.kernel,
        out_type=jax.ShapeDtypeStruct((NC, NP, W), jnp.float32),
        mesh=mesh,
        scratch_types=[
            pltpu.VMEM((nfd, CHD), jnp.int32),
            pltpu.VMEM((1, CHT), jnp.int32),
            pltpu.VMEM((CHD, W), jnp.float32),
            pltpu.VMEM_SHARED((NP, W), jnp.float32),
        ],
    )
    def deg_kernel(dst_hbm, dstt_hbm, ones_hbm, zeros_hbm, out_hbm,
                   didx, didxt, ones_v, acc):
        c = lax.axis_index("c")
        s = lax.axis_index("s")
        w = c * NS + s

        pltpu.sync_copy(dst_hbm.at[w], didx)
        pltpu.sync_copy(dstt_hbm.at[w], didxt)
        pltpu.sync_copy(ones_hbm, ones_v)
        pltpu.sync_copy(zeros_hbm, acc.at[pl.ds(s * RPT, RPT)])
        plsc.subcore_barrier()

        def chunk(i, _):
            pltpu.sync_copy(ones_v, acc.at[didx.at[i]], add=True)
            return 0

        lax.fori_loop(0, nfd, chunk, 0)
        pltpu.sync_copy(ones_v.at[pl.ds(0, CHT)], acc.at[didxt.at[0]], add=True)
        plsc.subcore_barrier()
        pltpu.sync_copy(acc.at[pl.ds(s * RPT, RPT)],
                        out_hbm.at[c, pl.ds(s * RPT, RPT)])

    return deg_kernel(dst3, dst3t, ones, zeros)


def _agg_partials(hp, src1, dst3, zeros):
    n_nodes, F = hp.shape
    _, nf, _ = dst3.shape
    assert nf % 2 == 1
    EPW = nf * CH
    NP = _pad_nodes(n_nodes)
    RPT = NP // NS
    mesh = plsc.VectorSubcoreMesh(core_axis_name="c", subcore_axis_name="s",
                                  num_cores=NC, num_subcores=NS)

    @functools.partial(
        pl.kernel,
        out_type=jax.ShapeDtypeStruct((NC, NP, F), jnp.float32),
        mesh=mesh,
        scratch_types=[
            pltpu.VMEM((EPW,), jnp.int32),
            pltpu.VMEM((nf, CH), jnp.int32),
            pltpu.VMEM((CH, F), jnp.float32),
            pltpu.VMEM((CH, F), jnp.float32),
            pltpu.VMEM_SHARED((NP, F), jnp.float32),
            pltpu.SemaphoreType.DMA,
            pltpu.SemaphoreType.DMA,
        ],
    )
    def agg_kernel(hp_hbm, src_hbm, dst_hbm, zeros_hbm, out_hbm,
                   sidx, didx, rows0, rows1, acc, sem0, sem1):
        c = lax.axis_index("c")
        s = lax.axis_index("s")
        w = c * NS + s

        pltpu.sync_copy(src_hbm.at[pl.ds(w * EPW, EPW)], sidx)
        pltpu.sync_copy(dst_hbm.at[w], didx)
        pltpu.async_copy(hp_hbm.at[sidx.at[pl.ds(0, CH)]], rows0, sem0)
        pltpu.async_copy(hp_hbm.at[sidx.at[pl.ds(CH, CH)]], rows1, sem1)
        pltpu.sync_copy(zeros_hbm, acc.at[pl.ds(s * RPT, RPT)])
        plsc.subcore_barrier()

        def half(i, rows, sem):
            pltpu.make_async_copy(hp_hbm.at[sidx.at[pl.ds(0, CH)]], rows, sem).wait()
            pltpu.sync_copy(rows, acc.at[didx.at[i]], add=True)

            @pl.when(i + 2 < nf)
            def _():
                pltpu.async_copy(hp_hbm.at[sidx.at[pl.ds((i + 2) * CH, CH)]], rows, sem)

        def pair(k, _):
            half(2 * k, rows0, sem0)
            half(2 * k + 1, rows1, sem1)
            return 0

        lax.fori_loop(0, (nf - 1) // 2, pair, 0)
        half(nf - 1, rows0, sem0)
        plsc.subcore_barrier()
        pltpu.sync_copy(acc.at[pl.ds(s * RPT, RPT)],
                        out_hbm.at[c, pl.ds(s * RPT, RPT)])

    return agg_kernel(hp, src1, dst3, zeros)


def _mm(x, W):
    n, F = x.shape
    H = W.shape[0]
    BR = 1000
    nb = n // BR

    def body(x_ref, w_ref, o_ref):
        o_ref[...] = lax.dot_general(x_ref[...], w_ref[...],
                                     (((1,), (1,)), ((), ())),
                                     preferred_element_type=jnp.float32)

    return pl.pallas_call(
        body,
        grid=(nb,),
        in_specs=[pl.BlockSpec((BR, F), lambda i: (i, 0)),
                  pl.BlockSpec((H, F), lambda i: (0, 0))],
        out_specs=pl.BlockSpec((BR, H), lambda i: (i, 0)),
        out_shape=jax.ShapeDtypeStruct((n, H), jnp.float32),
    )(x, W)


def _scale(h, deg0, deg1):
    n, F = h.shape
    BR = 1000
    nb = n // BR

    def body(h_ref, d0_ref, d1_ref, o_ref, dis_ref):
        dis = lax.rsqrt(d0_ref[:, 0:1] + d1_ref[:, 0:1] + 1.0)
        o_ref[...] = h_ref[...] * dis
        dis_ref[...] = dis

    return pl.pallas_call(
        body,
        grid=(nb,),
        in_specs=[pl.BlockSpec((BR, F), lambda i: (i, 0)),
                  pl.BlockSpec((BR, 128), lambda i: (i, 0)),
                  pl.BlockSpec((BR, 128), lambda i: (i, 0))],
        out_specs=[pl.BlockSpec((BR, F), lambda i: (i, 0)),
                   pl.BlockSpec((BR, 1), lambda i: (i, 0))],
        out_shape=[jax.ShapeDtypeStruct((n, F), jnp.float32),
                   jax.ShapeDtypeStruct((n, 1), jnp.float32)],
    )(h, deg0, deg1)


def _layer_mm(a0, a1, hp, dis1, b, W):
    n, F = hp.shape
    H = W.shape[0]
    BR = 1000
    nb = n // BR

    def body(a0_ref, a1_ref, hp_ref, dis_ref, b_ref, w_ref, o_ref):
        dis = dis_ref[...]
        z = jnp.maximum(dis * (a0_ref[...] + a1_ref[...] + hp_ref[...])
                        + b_ref[...], 0.0)
        h = lax.dot_general(z, w_ref[...], (((1,), (1,)), ((), ())),
                            preferred_element_type=jnp.float32)
        o_ref[...] = h * dis

    return pl.pallas_call(
        body,
        grid=(nb,),
        in_specs=[pl.BlockSpec((BR, F), lambda i: (i, 0)),
                  pl.BlockSpec((BR, F), lambda i: (i, 0)),
                  pl.BlockSpec((BR, F), lambda i: (i, 0)),
                  pl.BlockSpec((BR, 1), lambda i: (i, 0)),
                  pl.BlockSpec((1, F), lambda i: (0, 0)),
                  pl.BlockSpec((H, F), lambda i: (0, 0))],
        out_specs=pl.BlockSpec((BR, H), lambda i: (i, 0)),
        out_shape=jax.ShapeDtypeStruct((n, H), jnp.float32),
    )(a0, a1, hp, dis1, b, W)


def _pool_heads(a0, a1, hp, dis1, b, batch2d, Wv, bv, Wp, bp, n_graphs):
    n, F = hp.shape
    A = Wp.shape[0]
    G = n_graphs
    BR = 1000
    nb = n // BR

    def body(a0_ref, a1_ref, hp_ref, dis_ref, b_ref, bt_ref,
             wv_ref, bv_ref, wp_ref, bp_ref, v_ref, p_ref, pool_acc, cnt_acc):
        i = pl.program_id(0)

        @pl.when(i == 0)
        def _():
            pool_acc[...] = jnp.zeros_like(pool_acc)
            cnt_acc[...] = jnp.zeros_like(cnt_acc)

        dis = dis_ref[...]
        z = jnp.maximum(dis * (a0_ref[...] + a1_ref[...] + hp_ref[...])
                        + b_ref[...], 0.0)
        oh = (bt_ref[...] == lax.broadcasted_iota(jnp.int32, (1, G), 1)
              ).astype(jnp.float32)
        pool_acc[...] += lax.dot_general(oh, z, (((0,), (0,)), ((), ())),
                                         precision=lax.Precision.HIGHEST,
                                         preferred_element_type=jnp.float32)
        cnt_acc[...] += lax.dot_general(oh, jnp.ones((BR, F), jnp.float32),
                                        (((0,), (0,)), ((), ())),
                                        precision=lax.Precision.HIGHEST,
                                        preferred_element_type=jnp.float32)

        @pl.when(i == nb - 1)
        def _():
            pooled = pool_acc[...] / jnp.maximum(cnt_acc[...], 1.0)
            lv = lax.dot_general(pooled, wv_ref[...], (((1,), (1,)), ((), ())),
                                 preferred_element_type=jnp.float32)
            v_ref[...] = jnp.tanh(lv[:, 0:1] + bv_ref[0, 0])
            logits = lax.dot_general(pooled, wp_ref[...],
                                     (((1,), (1,)), ((), ())),
                                     preferred_element_type=jnp.float32) + bp_ref[...]
            m = jnp.max(logits, axis=1, keepdims=True)
            e = jnp.exp(logits - m)
            p_ref[...] = e / jnp.sum(e, axis=1, keepdims=True)

    return pl.pallas_call(
        body,
        grid=(nb,),
        in_specs=[pl.BlockSpec((BR, F), lambda i: (i, 0)),
                  pl.BlockSpec((BR, F), lambda i: (i, 0)),
                  pl.BlockSpec((BR, F), lambda i: (i, 0)),
                  pl.BlockSpec((BR, 1), lambda i: (i, 0)),
                  pl.BlockSpec((1, F), lambda i: (0, 0)),
                  pl.BlockSpec((BR, 1), lambda i: (i, 0)),
                  pl.BlockSpec((F, F), lambda i: (0, 0)),
                  pl.BlockSpec((1, 1), lambda i: (0, 0)),
                  pl.BlockSpec((A, F), lambda i: (0, 0)),
                  pl.BlockSpec((1, A), lambda i: (0, 0))],
        out_specs=[pl.BlockSpec((G, 1), lambda i: (0, 0)),
                   pl.BlockSpec((G, A), lambda i: (0, 0))],
        out_shape=[jax.ShapeDtypeStruct((G, 1), jnp.float32),
                   jax.ShapeDtypeStruct((G, A), jnp.float32)],
        scratch_shapes=[pltpu.VMEM((G, F), jnp.float32),
                        pltpu.VMEM((G, F), jnp.float32)],
    )(a0, a1, hp, dis1, b, batch2d, Wv, bv, Wp, bp)


def kernel(x, edge_index, batch, W1, b1, W2, b2, Wv, bv, Wp, bp):
    n = x.shape[0]
    E = edge_index.shape[1]
    nf = E // NW // CH
    src1 = edge_index[0]
    dst3 = edge_index[1].reshape(NW, nf, CH)
    EPW = E // NW
    nfd = EPW // 128
    dst2 = edge_index[1].reshape(NW, EPW)
    dst3d = dst2[:, :nfd * 128].reshape(NW, nfd, 128)
    dst3t = dst2[:, nfd * 128:].reshape(NW, 1, EPW - nfd * 128)
    RPT = _pad_nodes(n) // NS
    ones = jnp.ones((128, 128), jnp.float32)
    zeros = jnp.zeros((RPT, 128), jnp.float32)
    G = 64

    degp = _deg_partials(dst3d, dst3t, ones, zeros, n)
    h1 = _mm(x, W1)
    h1p, dis1 = _scale(h1, degp[0], degp[1])
    ag1 = _agg_partials(h1p, src1, dst3, zeros)
    h2p = _layer_mm(ag1[0], ag1[1], h1p, dis1, b1.reshape(1, -1), W2)
    ag2 = _agg_partials(h2p, src1, dst3, zeros)
    Wv_pad = jnp.zeros((x.shape[1], x.shape[1]), jnp.float32).at[0].set(Wv[0])
    v, p = _pool_heads(ag2[0], ag2[1], h2p, dis1, b2.reshape(1, -1),
                       batch.reshape(-1, 1), Wv_pad, bv.reshape(1, 1), Wp,
                       bp.reshape(1, -1), G)
    return (v, p)


# speedup vs baseline: 1.0085x; 1.0085x over previous
import functools

import jax
import jax.numpy as jnp
from jax import lax
from jax.experimental import pallas as pl
from jax.experimental.pallas import tpu as pltpu
from jax.experimental.pallas import tpu_sc as plsc

NC = 2
NS = 16
NW = NC * NS
CH = 80


def _pad_nodes(n_nodes):
    block = NS * 128
    return ((n_nodes + block - 1) // block) * block


def _deg_partials(dst3, ones, zeros, n_nodes):
    _, nf, _ = dst3.shape
    NP = _pad_nodes(n_nodes)
    RPT = NP // NS
    W = 128
    mesh = plsc.VectorSubcoreMesh(core_axis_name="c", subcore_axis_name="s",
                                  num_cores=NC, num_subcores=NS)

    @functools.partial(
        pl.kernel,
        out_type=jax.ShapeDtypeStruct((NC, NP, W), jnp.float32),
        mesh=mesh,
        scratch_types=[
            pltpu.VMEM((nf, CH), jnp.int32),
            pltpu.VMEM((CH, W), jnp.float32),
            pltpu.VMEM_SHARED((NP, W), jnp.float32),
        ],
    )
    def deg_kernel(dst_hbm, ones_hbm, zeros_hbm, out_hbm, didx, ones_v, acc):
        c = lax.axis_index("c")
        s = lax.axis_index("s")
        w = c * NS + s

        pltpu.sync_copy(dst_hbm.at[w], didx)
        pltpu.sync_copy(ones_hbm, ones_v)
        pltpu.sync_copy(zeros_hbm, acc.at[pl.ds(s * RPT, RPT)])
        plsc.subcore_barrier()

        def chunk(i, _):
            pltpu.sync_copy(ones_v, acc.at[didx.at[i]], add=True)
            return 0

        lax.fori_loop(0, nf, chunk, 0)
        plsc.subcore_barrier()
        pltpu.sync_copy(acc.at[pl.ds(s * RPT, RPT)],
                        out_hbm.at[c, pl.ds(s * RPT, RPT)])

    return deg_kernel(dst3, ones, zeros)


def _agg_partials(hp, src1, dst3, zeros):
    n_nodes, F = hp.shape
    _, nf, _ = dst3.shape
    assert nf % 2 == 1
    EPW = nf * CH
    NP = _pad_nodes(n_nodes)
    RPT = NP // NS
    mesh = plsc.VectorSubcoreMesh(core_axis_name="c", subcore_axis_name="s",
                                  num_cores=NC, num_subcores=NS)

    @functools.partial(
        pl.kernel,
        out_type=jax.ShapeDtypeStruct((NC, NP, F), jnp.float32),
        mesh=mesh,
        scratch_types=[
            pltpu.VMEM((EPW,), jnp.int32),
            pltpu.VMEM((nf, CH), jnp.int32),
            pltpu.VMEM((CH, F), jnp.float32),
            pltpu.VMEM((CH, F), jnp.float32),
            pltpu.VMEM_SHARED((NP, F), jnp.float32),
            pltpu.SemaphoreType.DMA,
            pltpu.SemaphoreType.DMA,
        ],
    )
    def agg_kernel(hp_hbm, src_hbm, dst_hbm, zeros_hbm, out_hbm,
                   sidx, didx, rows0, rows1, acc, sem0, sem1):
        c = lax.axis_index("c")
        s = lax.axis_index("s")
        w = c * NS + s

        pltpu.sync_copy(src_hbm.at[pl.ds(w * EPW, EPW)], sidx)
        pltpu.sync_copy(dst_hbm.at[w], didx)
        pltpu.async_copy(hp_hbm.at[sidx.at[pl.ds(0, CH)]], rows0, sem0)
        pltpu.async_copy(hp_hbm.at[sidx.at[pl.ds(CH, CH)]], rows1, sem1)
        pltpu.sync_copy(zeros_hbm, acc.at[pl.ds(s * RPT, RPT)])
        plsc.subcore_barrier()

        def half(i, rows, sem):
            pltpu.make_async_copy(hp_hbm.at[sidx.at[pl.ds(0, CH)]], rows, sem).wait()
            pltpu.sync_copy(rows, acc.at[didx.at[i]], add=True)

            @pl.when(i + 2 < nf)
            def _():
                pltpu.async_copy(hp_hbm.at[sidx.at[pl.ds((i + 2) * CH, CH)]], rows, sem)

        def pair(k, _):
            half(2 * k, rows0, sem0)
            half(2 * k + 1, rows1, sem1)
            return 0

        lax.fori_loop(0, (nf - 1) // 2, pair, 0)
        half(nf - 1, rows0, sem0)
        plsc.subcore_barrier()
        pltpu.sync_copy(acc.at[pl.ds(s * RPT, RPT)],
                        out_hbm.at[c, pl.ds(s * RPT, RPT)])

    return agg_kernel(hp, src1, dst3, zeros)


def _mm_scale(x, W, deg0, deg1):
    n, F = x.shape
    H = W.shape[0]
    BR = 1000
    nb = n // BR

    def body(x_ref, w_ref, d0_ref, d1_ref, o_ref, dis_ref):
        dis = lax.rsqrt(d0_ref[:, 0:1] + d1_ref[:, 0:1] + 1.0)
        h = lax.dot_general(x_ref[...], w_ref[...], (((1,), (1,)), ((), ())),
                            preferred_element_type=jnp.float32)
        o_ref[...] = h * dis
        dis_ref[...] = dis

    return pl.pallas_call(
        body,
        grid=(nb,),
        in_specs=[pl.BlockSpec((BR, F), lambda i: (i, 0)),
                  pl.BlockSpec((H, F), lambda i: (0, 0)),
                  pl.BlockSpec((BR, 128), lambda i: (i, 0)),
                  pl.BlockSpec((BR, 128), lambda i: (i, 0))],
        out_specs=[pl.BlockSpec((BR, H), lambda i: (i, 0)),
                   pl.BlockSpec((BR, 1), lambda i: (i, 0))],
        out_shape=[jax.ShapeDtypeStruct((n, H), jnp.float32),
                   jax.ShapeDtypeStruct((n, 1), jnp.float32)],
    )(x, W, deg0, deg1)


def _layer_mm(a0, a1, hp, dis1, b, W):
    n, F = hp.shape
    H = W.shape[0]
    BR = 1000
    nb = n // BR

    def body(a0_ref, a1_ref, hp_ref, dis_ref, b_ref, w_ref, o_ref):
        dis = dis_ref[...]
        z = jnp.maximum(dis * (a0_ref[...] + a1_ref[...] + hp_ref[...])
                        + b_ref[...], 0.0)
        h = lax.dot_general(z, w_ref[...], (((1,), (1,)), ((), ())),
                            preferred_element_type=jnp.float32)
        o_ref[...] = h * dis

    return pl.pallas_call(
        body,
        grid=(nb,),
        in_specs=[pl.BlockSpec((BR, F), lambda i: (i, 0)),
                  pl.BlockSpec((BR, F), lambda i: (i, 0)),
                  pl.BlockSpec((BR, F), lambda i: (i, 0)),
                  pl.BlockSpec((BR, 1), lambda i: (i, 0)),
                  pl.BlockSpec((1, F), lambda i: (0, 0)),
                  pl.BlockSpec((H, F), lambda i: (0, 0))],
        out_specs=pl.BlockSpec((BR, H), lambda i: (i, 0)),
        out_shape=jax.ShapeDtypeStruct((n, H), jnp.float32),
    )(a0, a1, hp, dis1, b, W)


def _pool_heads(a0, a1, hp, dis1, b, batch2d, Wv, bv, Wp, bp, n_graphs):
    n, F = hp.shape
    A = Wp.shape[0]
    G = n_graphs
    BR = 1000
    nb = n // BR

    def body(a0_ref, a1_ref, hp_ref, dis_ref, b_ref, bt_ref,
             wv_ref, bv_ref, wp_ref, bp_ref, v_ref, p_ref, pool_acc, cnt_acc):
        i = pl.program_id(0)

        @pl.when(i == 0)
        def _():
            pool_acc[...] = jnp.zeros_like(pool_acc)
            cnt_acc[...] = jnp.zeros_like(cnt_acc)

        dis = dis_ref[...]
        z = jnp.maximum(dis * (a0_ref[...] + a1_ref[...] + hp_ref[...])
                        + b_ref[...], 0.0)
        oh = (bt_ref[...] == lax.broadcasted_iota(jnp.int32, (1, G), 1)
              ).astype(jnp.float32)
        pool_acc[...] += lax.dot_general(oh, z, (((0,), (0,)), ((), ())),
                                         precision=lax.Precision.HIGHEST,
                                         preferred_element_type=jnp.float32)
        cnt_acc[...] += lax.dot_general(oh, jnp.ones((BR, F), jnp.float32),
                                        (((0,), (0,)), ((), ())),
                                        precision=lax.Precision.HIGHEST,
                                        preferred_element_type=jnp.float32)

        @pl.when(i == nb - 1)
        def _():
            pooled = pool_acc[...] / jnp.maximum(cnt_acc[...], 1.0)
            lv = lax.dot_general(pooled, wv_ref[...], (((1,), (1,)), ((), ())),
                                 preferred_element_type=jnp.float32)
            v_ref[...] = jnp.tanh(lv[:, 0:1] + bv_ref[0, 0])
            logits = lax.dot_general(pooled, wp_ref[...],
                                     (((1,), (1,)), ((), ())),
                                     preferred_element_type=jnp.float32) + bp_ref[...]
            m = jnp.max(logits, axis=1, keepdims=True)
            e = jnp.exp(logits - m)
            p_ref[...] = e / jnp.sum(e, axis=1, keepdims=True)

    return pl.pallas_call(
        body,
        grid=(nb,),
        in_specs=[pl.BlockSpec((BR, F), lambda i: (i, 0)),
                  pl.BlockSpec((BR, F), lambda i: (i, 0)),
                  pl.BlockSpec((BR, F), lambda i: (i, 0)),
                  pl.BlockSpec((BR, 1), lambda i: (i, 0)),
                  pl.BlockSpec((1, F), lambda i: (0, 0)),
                  pl.BlockSpec((BR, 1), lambda i: (i, 0)),
                  pl.BlockSpec((F, F), lambda i: (0, 0)),
                  pl.BlockSpec((1, 1), lambda i: (0, 0)),
                  pl.BlockSpec((A, F), lambda i: (0, 0)),
                  pl.BlockSpec((1, A), lambda i: (0, 0))],
        out_specs=[pl.BlockSpec((G, 1), lambda i: (0, 0)),
                   pl.BlockSpec((G, A), lambda i: (0, 0))],
        out_shape=[jax.ShapeDtypeStruct((G, 1), jnp.float32),
                   jax.ShapeDtypeStruct((G, A), jnp.float32)],
        scratch_shapes=[pltpu.VMEM((G, F), jnp.float32),
                        pltpu.VMEM((G, F), jnp.float32)],
    )(a0, a1, hp, dis1, b, batch2d, Wv, bv, Wp, bp)


def kernel(x, edge_index, batch, W1, b1, W2, b2, Wv, bv, Wp, bp):
    n = x.shape[0]
    E = edge_index.shape[1]
    nf = E // NW // CH
    src1 = edge_index[0]
    dst3 = edge_index[1].reshape(NW, nf, CH)
    RPT = _pad_nodes(n) // NS
    ones = jnp.ones((CH, 128), jnp.float32)
    zeros = jnp.zeros((RPT, 128), jnp.float32)
    G = 64

    degp = _deg_partials(dst3, ones, zeros, n)
    h1p, dis1 = _mm_scale(x, W1, degp[0], degp[1])
    ag1 = _agg_partials(h1p, src1, dst3, zeros)
    h2p = _layer_mm(ag1[0], ag1[1], h1p, dis1, b1.reshape(1, -1), W2)
    ag2 = _agg_partials(h2p, src1, dst3, zeros)
    Wv_pad = jnp.zeros((x.shape[1], x.shape[1]), jnp.float32).at[0].set(Wv[0])
    v, p = _pool_heads(ag2[0], ag2[1], h2p, dis1, b2.reshape(1, -1),
                       batch.reshape(-1, 1), Wv_pad, bv.reshape(1, 1), Wp,
                       bp.reshape(1, -1), G)
    return (v, p)
